# baseline (speedup 1.0000x reference)
import functools

import jax
import jax.numpy as jnp
from jax import lax
from jax.experimental import pallas as pl
from jax.experimental.pallas import tpu as pltpu
from jax.experimental.pallas import tpu_sc as plsc

_NC = 2
_NS = 16
_CH = 128


def _linear_body(x_ref, w_ref, b_ref, o_ref):
    o_ref[...] = (
        jnp.dot(x_ref[...], w_ref[...], preferred_element_type=jnp.float32)
        + b_ref[...]
    )


def _sum_linear_body(a_ref, a2_ref, w_ref, b_ref, o_ref):
    a = a_ref[...] + a2_ref[...]
    o_ref[...] = (
        jnp.dot(a, w_ref[...], preferred_element_type=jnp.float32) + b_ref[...]
    )


def _block_rows(m):
    for bm in (1024, 1000, 512, 500, 256, 250, 128, 64, 32, 16, 8):
        if m % bm == 0:
            return bm
    return m


def _linear(x, w, b, body, extra=None):
    m, d = x.shape
    dout = w.shape[1]
    bm = _block_rows(m)
    xs = [x] if extra is None else [x, extra]
    in_specs = [pl.BlockSpec((bm, d), lambda i: (i, 0)) for _ in xs]
    in_specs += [
        pl.BlockSpec((d, dout), lambda i: (0, 0)),
        pl.BlockSpec((1, dout), lambda i: (0, 0)),
    ]
    return pl.pallas_call(
        body,
        grid=(m // bm,),
        in_specs=in_specs,
        out_specs=pl.BlockSpec((bm, dout), lambda i: (i, 0)),
        out_shape=jax.ShapeDtypeStruct((m, dout), jnp.float32),
    )(*xs, w, b.reshape(1, dout))


@functools.cache
def _make_sc_scatter(n_acc, ep, d):
    nw = _NC * _NS
    nchunks = ep // (nw * _CH)
    assert nchunks % 8 == 0
    rows_per_sub = n_acc // _NS
    mesh = plsc.VectorSubcoreMesh(core_axis_name="c", subcore_axis_name="s")

    @functools.partial(
        pl.kernel,
        mesh=mesh,
        out_type=jax.ShapeDtypeStruct((_NC, n_acc, d), jnp.float32),
        scratch_types=[
            pltpu.VMEM((nchunks, _CH), jnp.int32),
            pltpu.VMEM((nchunks, _CH), jnp.int32),
            pltpu.VMEM((_CH, d), jnp.float32),
            pltpu.VMEM_SHARED((n_acc, d), jnp.float32),
            pltpu.SemaphoreType.DMA,
        ],
    )
    def sc_scatter(h_hbm, src_hbm, dst_hbm, z_hbm, out_hbm,
                   src_v, dst_v, rows_v, acc_sh, sem):
        c = lax.axis_index("c")
        s = lax.axis_index("s")
        wid = s * _NC + c
        r0 = s * rows_per_sub
        pltpu.sync_copy(src_hbm.at[pl.ds(wid * nchunks, nchunks)], src_v)
        pltpu.sync_copy(dst_hbm.at[pl.ds(wid * nchunks, nchunks)], dst_v)
        pltpu.sync_copy(z_hbm, acc_sh.at[pl.ds(r0, rows_per_sub)])
        plsc.subcore_barrier()

        def body(j, carry):
            pltpu.async_copy(h_hbm.at[src_v.at[j]], rows_v, sem).wait()
            pltpu.sync_copy(rows_v, acc_sh.at[dst_v.at[j]], add=True)
            return carry

        lax.fori_loop(0, nchunks, body, 0)

        plsc.subcore_barrier()
        pltpu.sync_copy(acc_sh.at[pl.ds(r0, rows_per_sub)],
                        out_hbm.at[c, pl.ds(r0, rows_per_sub)])

    return sc_scatter


def kernel(x, edge_index, W_pre, b_pre, W_post, b_post):
    n, d = x.shape
    e = edge_index.shape[1]
    src = edge_index[0].astype(jnp.int32)
    dst = edge_index[1].astype(jnp.int32)

    nw = _NC * _NS
    quant = nw * _CH * 8
    ep = ((e + quant - 1) // quant) * quant
    n_acc = ((n + 1 + 1023) // 1024) * 1024
    pad = ep - e
    if pad:
        src = jnp.concatenate([src, jnp.zeros((pad,), jnp.int32)])
        pad_dst = n + jnp.arange(pad, dtype=jnp.int32) % (n_acc - n)
        dst = jnp.concatenate([dst, pad_dst])

    h = _linear(x, W_pre, b_pre, _linear_body)
    z = jnp.zeros((n_acc // _NS, d), jnp.float32)
    parts = _make_sc_scatter(n_acc, ep, d)(
        h, src.reshape(ep // _CH, _CH), dst.reshape(ep // _CH, _CH), z)
    out = _linear(parts[0], W_post, b_post, _sum_linear_body, extra=parts[1])
    return out[:n]


# speedup vs baseline: 1.0446x; 1.0446x over previous
import functools

import jax
import jax.numpy as jnp
from jax import lax
from jax.experimental import pallas as pl
from jax.experimental.pallas import tpu as pltpu
from jax.experimental.pallas import tpu_sc as plsc

_NC = 2
_NS = 16
_CH = 128


def _linear_body(x_ref, w_ref, b_ref, o_ref):
    o_ref[...] = (
        jnp.dot(x_ref[...], w_ref[...], preferred_element_type=jnp.float32)
        + b_ref[...]
    )


def _sum_linear_body(a_ref, a2_ref, w_ref, b_ref, o_ref):
    a = a_ref[...] + a2_ref[...]
    o_ref[...] = (
        jnp.dot(a, w_ref[...], preferred_element_type=jnp.float32) + b_ref[...]
    )


def _block_rows(m):
    for bm in (1024, 1000, 512, 500, 256, 250, 128, 64, 32, 16, 8):
        if m % bm == 0:
            return bm
    return m


def _linear(x, w, b, body, extra=None):
    m, d = x.shape
    dout = w.shape[1]
    bm = _block_rows(m)
    xs = [x] if extra is None else [x, extra]
    in_specs = [pl.BlockSpec((bm, d), lambda i: (i, 0)) for _ in xs]
    in_specs += [
        pl.BlockSpec((d, dout), lambda i: (0, 0)),
        pl.BlockSpec((1, dout), lambda i: (0, 0)),
    ]
    return pl.pallas_call(
        body,
        grid=(m // bm,),
        in_specs=in_specs,
        out_specs=pl.BlockSpec((bm, dout), lambda i: (i, 0)),
        out_shape=jax.ShapeDtypeStruct((m, dout), jnp.float32),
    )(*xs, w, b.reshape(1, dout))


@functools.cache
def _make_sc_scatter(n_acc, ep, d):
    nw = _NC * _NS
    nchunks = ep // (nw * _CH)
    assert nchunks % 8 == 0
    rows_per_sub = n_acc // _NS
    mesh = plsc.VectorSubcoreMesh(core_axis_name="c", subcore_axis_name="s")

    @functools.partial(
        pl.kernel,
        mesh=mesh,
        out_type=jax.ShapeDtypeStruct((_NC, n_acc, d), jnp.float32),
        scratch_types=[
            pltpu.VMEM((2, _CH), jnp.int32),
            pltpu.VMEM((2, _CH), jnp.int32),
            pltpu.VMEM((2, _CH, d), jnp.float32),
            pltpu.VMEM_SHARED((n_acc, d), jnp.float32),
        ]
        + [pltpu.SemaphoreType.DMA] * 4,
    )
    def sc_scatter(h_hbm, src_hbm, dst_hbm, z_hbm, out_hbm,
                   src_v, dst_v, rows_v, acc_sh, *sems):
        isem = sems[:2]
        gsem = sems[2:]
        c = lax.axis_index("c")
        s = lax.axis_index("s")
        wid = s * _NC + c
        r0 = s * rows_per_sub
        base = wid * nchunks * _CH

        def idx_load(j, b, sync):
            srcs = src_hbm.at[pl.ds(base + j * _CH, _CH)]
            dsts = dst_hbm.at[pl.ds(base + j * _CH, _CH)]
            if sync:
                pltpu.sync_copy(srcs, src_v.at[b])
                pltpu.sync_copy(dsts, dst_v.at[b])
            else:
                pltpu.async_copy(srcs, src_v.at[b], isem[b])
                pltpu.async_copy(dsts, dst_v.at[b], isem[b])

        def idx_wait(b):
            for _ in range(2):
                pltpu.make_async_copy(src_hbm.at[pl.ds(0, _CH)],
                                      src_v.at[b], isem[b]).wait()

        def gather(j_unused, b):
            pltpu.async_copy(h_hbm.at[src_v.at[b]], rows_v.at[b], gsem[b])

        def gwait(b):
            pltpu.make_async_copy(h_hbm.at[pl.ds(0, _CH)],
                                  rows_v.at[b], gsem[b]).wait()

        def scat(b):
            pltpu.sync_copy(rows_v.at[b], acc_sh.at[dst_v.at[b]], add=True)

        pltpu.sync_copy(z_hbm, acc_sh.at[pl.ds(r0, rows_per_sub)])
        plsc.subcore_barrier()

        idx_load(0, 0, sync=True)
        gather(0, 0)
        idx_load(1, 1, sync=False)

        def group(g, carry):
            for b in range(2):
                j = 2 * g + b
                gwait(b)
                idx_wait(1 - b)
                gather(j + 1, 1 - b)
                scat(b)
                idx_load(j + 2, b, sync=False)
            return carry

        lax.fori_loop(0, (nchunks - 2) // 2, group, 0)
        gwait(0)
        idx_wait(1)
        gather(nchunks - 1, 1)
        scat(0)
        gwait(1)
        scat(1)

        plsc.subcore_barrier()
        pltpu.sync_copy(acc_sh.at[pl.ds(r0, rows_per_sub)],
                        out_hbm.at[c, pl.ds(r0, rows_per_sub)])

    return sc_scatter


def kernel(x, edge_index, W_pre, b_pre, W_post, b_post):
    n, d = x.shape
    e = edge_index.shape[1]
    src = edge_index[0].astype(jnp.int32)
    dst = edge_index[1].astype(jnp.int32)

    nw = _NC * _NS
    quant = nw * _CH * 8
    ep = ((e + quant - 1) // quant) * quant
    n_acc = ((n + 1 + 1023) // 1024) * 1024
    pad = ep - e
    if pad:
        src = jnp.concatenate([src, jnp.zeros((pad,), jnp.int32)])
        pad_dst = n + jnp.arange(pad, dtype=jnp.int32) % (n_acc - n)
        dst = jnp.concatenate([dst, pad_dst])

    h = _linear(x, W_pre, b_pre, _linear_body)
    z = jnp.zeros((n_acc // _NS, d), jnp.float32)
    parts = _make_sc_scatter(n_acc, ep, d)(h, src, dst, z)
    out = _linear(parts[0], W_post, b_post, _sum_linear_body, extra=parts[1])
    return out[:n]
